# baseline (speedup 1.0000x reference)
import functools

import jax
import jax.numpy as jnp
from jax import lax
from jax.experimental import pallas as pl
from jax.experimental.pallas import tpu as pltpu


def _body(ts_ref, emb_ref, out_ref, *, rows, seq):
    ts = ts_ref[...]
    tj = ts[:, :, None]
    tk = ts[:, None, :]
    jj = lax.broadcasted_iota(jnp.int32, (rows, seq, seq), 1)
    kk = lax.broadcasted_iota(jnp.int32, (rows, seq, seq), 2)
    smaller = (tk < tj) | ((tk == tj) & (kk < jj))
    rank = jnp.sum(jnp.where(smaller, 1, 0), axis=2)
    ii = lax.broadcasted_iota(jnp.int32, (rows, seq, seq), 1)
    onehot = jnp.where(rank[:, None, :] == ii, 1.0, 0.0)
    emb = emb_ref[...]
    for r in range(rows):
        out_ref[r, :, :] = jnp.dot(onehot[r], emb,
                                   preferred_element_type=jnp.float32)


def kernel(ts, pos_embeddings):
    batch, seq = ts.shape
    seq_t, dim = pos_embeddings.shape
    rows = 8
    grid = (batch // rows,)
    return pl.pallas_call(
        functools.partial(_body, rows=rows, seq=seq),
        grid=grid,
        in_specs=[
            pl.BlockSpec((rows, seq), lambda b: (b, 0)),
            pl.BlockSpec((seq_t, dim), lambda b: (0, 0)),
        ],
        out_specs=pl.BlockSpec((rows, seq, dim), lambda b: (b, 0, 0)),
        out_shape=jax.ShapeDtypeStruct((batch, seq, dim), jnp.float32),
    )(ts, pos_embeddings)


# speedup vs baseline: 15.4563x; 15.4563x over previous
import functools

import jax
import jax.numpy as jnp
from jax import lax
from jax.experimental import pallas as pl
from jax.experimental.pallas import tpu as pltpu
from jax.experimental.pallas import tpu_sc as plsc


def _rank_body(ts_ref, idx_ref, *, rows, seq):
    ts = ts_ref[...]
    tj = ts[:, :, None]
    tk = ts[:, None, :]
    jj = lax.broadcasted_iota(jnp.int32, (rows, seq, seq), 1)
    kk = lax.broadcasted_iota(jnp.int32, (rows, seq, seq), 2)
    smaller = (tk < tj) | ((tk == tj) & (kk < jj))
    rank = jnp.sum(jnp.where(smaller, 1, 0), axis=2)
    base = pl.program_id(0) * rows
    r_iota = lax.broadcasted_iota(jnp.int32, (rows, seq), 0)
    idx_ref[...] = rank + (base + r_iota) * seq


def _scatter_indices(ts, rows=8):
    batch, seq = ts.shape
    return pl.pallas_call(
        functools.partial(_rank_body, rows=rows, seq=seq),
        grid=(batch // rows,),
        in_specs=[pl.BlockSpec((rows, seq), lambda b: (b, 0))],
        out_specs=pl.BlockSpec((rows, seq), lambda b: (b, 0)),
        out_shape=jax.ShapeDtypeStruct((batch, seq), jnp.int32),
    )(ts)


_NC, _NS = 2, 16
_NW = _NC * _NS
_CH = 8


def _sc_scatter(idx, emb):
    batch, seq = idx.shape
    dim = emb.shape[-1]
    half = seq // 2
    bpw = batch // _NW
    nch = bpw // _CH
    idx3 = idx.reshape(batch * 2, half)
    emb3 = emb.reshape(2, half, dim)

    mesh = plsc.VectorSubcoreMesh(core_axis_name="c", subcore_axis_name="s")

    @functools.partial(
        pl.kernel,
        mesh=mesh,
        compiler_params=pltpu.CompilerParams(use_tc_tiling_on_sc=False),
        out_type=jax.ShapeDtypeStruct((batch * seq, dim), jnp.float32),
        scratch_types=[
            pltpu.VMEM((2, half, dim), jnp.float32),
            pltpu.VMEM((2, 2 * _CH, half), jnp.int32),
            pltpu.SemaphoreType.DMA,
            pltpu.SemaphoreType.DMA,
            pltpu.SemaphoreType.DMA,
        ],
    )
    def sck(idx_hbm, emb_hbm, out_hbm, table_v, idx_v, sem_tab, sem_idx,
            sem_out):
        wid = lax.axis_index("s") * _NC + lax.axis_index("c")
        row0 = wid * bpw * 2

        tab = pltpu.async_copy(emb_hbm, table_v, sem_tab)
        idx0 = pltpu.async_copy(
            idx_hbm.at[pl.ds(row0, 2 * _CH)], idx_v.at[0], sem_idx)
        tab.wait()
        idx0.wait()

        def chunk(c, _):
            buf = lax.rem(c, 2)
            scats = [
                pltpu.async_copy(
                    table_v.at[q % 2],
                    out_hbm.at[idx_v.at[buf, q]], sem_out)
                for q in range(2 * _CH)
            ]

            @pl.when(c + 1 < nch)
            def _prefetch():
                nxt = pltpu.async_copy(
                    idx_hbm.at[pl.ds(row0 + (c + 1) * 2 * _CH, 2 * _CH)],
                    idx_v.at[1 - buf], sem_idx)
                nxt.wait()

            for s in scats:
                s.wait()
            return _

        lax.fori_loop(0, nch, chunk, None)

    return sck(idx3, emb3)


def kernel(ts, pos_embeddings):
    batch, seq = ts.shape
    dim = pos_embeddings.shape[-1]
    idx = _scatter_indices(ts)
    out = _sc_scatter(idx, pos_embeddings)
    return out.reshape(batch, seq, dim)


# speedup vs baseline: 16.4997x; 1.0675x over previous
import functools

import jax
import jax.numpy as jnp
from jax import lax
from jax.experimental import pallas as pl
from jax.experimental.pallas import tpu as pltpu
from jax.experimental.pallas import tpu_sc as plsc


def _rank_body(ts_ref, idx_ref, *, rows, seq):
    ts = ts_ref[...]
    tj = ts[:, :, None]
    tk = ts[:, None, :]
    jj = lax.broadcasted_iota(jnp.int32, (seq, seq), 0)
    kk = lax.broadcasted_iota(jnp.int32, (seq, seq), 1)
    tri = (kk < jj)[None]
    smaller = (tk < tj) | ((tk == tj) & tri)
    rank = jnp.sum(jnp.where(smaller, 1.0, 0.0), axis=2)
    base = pl.program_id(0) * rows
    r_iota = lax.broadcasted_iota(jnp.int32, (rows, seq), 0)
    idx_ref[...] = rank.astype(jnp.int32) + (base + r_iota) * seq


def _scatter_indices(ts, rows=8):
    batch, seq = ts.shape
    return pl.pallas_call(
        functools.partial(_rank_body, rows=rows, seq=seq),
        grid=(batch // rows,),
        in_specs=[pl.BlockSpec((rows, seq), lambda b: (b, 0))],
        out_specs=pl.BlockSpec((rows, seq), lambda b: (b, 0)),
        out_shape=jax.ShapeDtypeStruct((batch, seq), jnp.int32),
    )(ts)


_NC, _NS = 2, 16
_NW = _NC * _NS
_CH = 8


def _sc_scatter(idx, emb):
    batch, seq = idx.shape
    dim = emb.shape[-1]
    half = seq // 2
    bpw = batch // _NW
    nch = bpw // _CH
    idx3 = idx.reshape(batch * 2, half)
    emb3 = emb.reshape(2, half, dim)

    mesh = plsc.VectorSubcoreMesh(core_axis_name="c", subcore_axis_name="s")

    @functools.partial(
        pl.kernel,
        mesh=mesh,
        compiler_params=pltpu.CompilerParams(use_tc_tiling_on_sc=False),
        out_type=jax.ShapeDtypeStruct((batch * seq, dim), jnp.float32),
        scratch_types=[
            pltpu.VMEM((2, half, dim), jnp.float32),
            pltpu.VMEM((2, 2 * _CH, half), jnp.int32),
            pltpu.SemaphoreType.DMA,
            pltpu.SemaphoreType.DMA,
            pltpu.SemaphoreType.DMA,
        ],
    )
    def sck(idx_hbm, emb_hbm, out_hbm, table_v, idx_v, sem_tab, sem_idx,
            sem_out):
        wid = lax.axis_index("s") * _NC + lax.axis_index("c")
        row0 = wid * bpw * 2

        tab = pltpu.async_copy(emb_hbm, table_v, sem_tab)
        idx0 = pltpu.async_copy(
            idx_hbm.at[pl.ds(row0, 2 * _CH)], idx_v.at[0], sem_idx)
        tab.wait()
        idx0.wait()

        def chunk(c, _):
            buf = lax.rem(c, 2)
            scats = [
                pltpu.async_copy(
                    table_v.at[q % 2],
                    out_hbm.at[idx_v.at[buf, q]], sem_out)
                for q in range(2 * _CH)
            ]

            @pl.when(c + 1 < nch)
            def _prefetch():
                nxt = pltpu.async_copy(
                    idx_hbm.at[pl.ds(row0 + (c + 1) * 2 * _CH, 2 * _CH)],
                    idx_v.at[1 - buf], sem_idx)
                nxt.wait()

            for s in scats:
                s.wait()
            return _

        lax.fori_loop(0, nch, chunk, None)

    return sck(idx3, emb3)


def kernel(ts, pos_embeddings):
    batch, seq = ts.shape
    dim = pos_embeddings.shape[-1]
    idx = _scatter_indices(ts)
    out = _sc_scatter(idx, pos_embeddings)
    return out.reshape(batch, seq, dim)
